# baseline (speedup 1.0000x reference)
import jax
import jax.numpy as jnp
from jax import lax
from jax.experimental import pallas as pl
from jax.experimental.pallas import tpu as pltpu
from jax.experimental.pallas import tpu_sc as plsc

_D = 1024
_LANES = 16
_NC, _NS = 2, 16
_NW = _NC * _NS
_CHUNK = 8
_RB = 8
_XB = 4
_LG = 4
_LX = 2


def _pe_body(x_hbm, pos_hbm, tab_hbm, out_hbm,
             idx_all, rowbuf, xbuf, *sems):
    gsems = sems[0:_RB]
    osems = sems[_RB:2 * _RB]
    xsems = sems[2 * _RB:2 * _RB + _XB]
    wid = lax.axis_index("s") * _NC + lax.axis_index("c")
    n_rows = pos_hbm.shape[0]
    rows_per_w = n_rows // _NW
    base_w = wid * rows_per_w
    n_chunks = rows_per_w // _CHUNK

    pltpu.sync_copy(pos_hbm.at[pl.ds(base_w, rows_per_w)], idx_all)

    def start_gather(c, slot):
        idx = idx_all.at[pl.ds(c * _CHUNK, _CHUNK)]
        pltpu.async_copy(tab_hbm.at[idx], rowbuf.at[slot], gsems[slot])

    def wait_gather(c, slot):
        idx = idx_all.at[pl.ds(c * _CHUNK, _CHUNK)]
        pltpu.make_async_copy(tab_hbm.at[idx], rowbuf.at[slot],
                              gsems[slot]).wait()

    def start_x(c, slot):
        pltpu.async_copy(x_hbm.at[pl.ds(base_w + c * _CHUNK, _CHUNK)],
                         xbuf.at[slot], xsems[slot])

    def wait_x(c, slot):
        pltpu.make_async_copy(x_hbm.at[pl.ds(base_w + c * _CHUNK, _CHUNK)],
                              xbuf.at[slot], xsems[slot]).wait()

    def start_out(c, slot):
        pltpu.async_copy(rowbuf.at[slot],
                         out_hbm.at[pl.ds(base_w + c * _CHUNK, _CHUNK)],
                         osems[slot])

    def wait_out(c, slot):
        pltpu.make_async_copy(rowbuf.at[slot],
                              out_hbm.at[pl.ds(base_w + c * _CHUNK, _CHUNK)],
                              osems[slot]).wait()

    def accumulate(gslot, xslot):
        @plsc.parallel_loop(0, _CHUNK, step=1, unroll=2)
        def _rows(r):
            for j in range(_D // _LANES):
                off = j * _LANES
                v = xbuf[xslot, r, pl.ds(off, _LANES)]
                plsc.addupdate(rowbuf.at[gslot, r, pl.ds(off, _LANES)], v)

    for c in range(_LG):
        start_gather(c, c % _RB)
    for c in range(_LX):
        start_x(c, c % _XB)

    def outer(i0, carry):
        for u in range(_RB):
            j = i0 * _RB + u
            gslot = u
            xslot = u % _XB
            wait_gather(j, gslot)
            wait_x(j, xslot)
            accumulate(gslot, xslot)
            start_out(j, gslot)

            @pl.when(j + _LX < n_chunks)
            def _issue_x():
                start_x(j + _LX, (u + _LX) % _XB)

            @pl.when(j >= _RB - _LG)
            def _drain():
                wait_out(j - (_RB - _LG), (u + _LG) % _RB)

            @pl.when(j + _LG < n_chunks)
            def _issue_g():
                start_gather(j + _LG, (u + _LG) % _RB)
        return carry

    lax.fori_loop(0, n_chunks // _RB, outer, 0)

    for c in range(n_chunks - (_RB - _LG), n_chunks):
        wait_out(c, c % _RB)


def kernel(x, positions, pos_table):
    b, s, d = x.shape
    n = b * s
    x2 = x.reshape(n, d)
    pos = positions.reshape(n).astype(jnp.int32)
    mesh = plsc.VectorSubcoreMesh(core_axis_name="c", subcore_axis_name="s")
    f = pl.kernel(
        _pe_body,
        mesh=mesh,
        out_type=jax.ShapeDtypeStruct((n, d), jnp.float32),
        scratch_types=[
            pltpu.VMEM((n // _NW,), jnp.int32),
            pltpu.VMEM((_RB, _CHUNK, d), jnp.float32),
            pltpu.VMEM((_XB, _CHUNK, d), jnp.float32),
        ] + [pltpu.SemaphoreType.DMA] * (2 * _RB + _XB),
    )
    out = f(x2, pos, pos_table)
    return out.reshape(b, s, d)


# speedup vs baseline: 1.5225x; 1.5225x over previous
import jax
import jax.numpy as jnp
from jax import lax
from jax.experimental import pallas as pl
from jax.experimental.pallas import tpu as pltpu
from jax.experimental.pallas import tpu_sc as plsc

_D = 1024
_LANES = 16
_NC, _NS = 2, 16
_NW = _NC * _NS
_CHUNK = 8
_NBUF = 4
_LEAD = 2


def _pe_body(x_hbm, pos_hbm, tab_hbm, out_hbm,
             idx_all, rowbuf, xbuf, *sems):
    gsems = sems[0:_NBUF]
    xsems = sems[_NBUF:2 * _NBUF]
    osems = sems[2 * _NBUF:3 * _NBUF]
    wid = lax.axis_index("s") * _NC + lax.axis_index("c")
    n_rows = pos_hbm.shape[0]
    rows_per_w = n_rows // _NW
    base_w = wid * rows_per_w
    n_chunks = rows_per_w // _CHUNK

    pltpu.sync_copy(pos_hbm.at[pl.ds(base_w, rows_per_w)], idx_all)

    def start_gather(c, slot):
        idx = idx_all.at[pl.ds(c * _CHUNK, _CHUNK)]
        pltpu.async_copy(tab_hbm.at[idx], rowbuf.at[slot], gsems[slot])

    def wait_gather(c, slot):
        idx = idx_all.at[pl.ds(c * _CHUNK, _CHUNK)]
        pltpu.make_async_copy(tab_hbm.at[idx], rowbuf.at[slot],
                              gsems[slot]).wait()

    def start_x(c, slot):
        pltpu.async_copy(x_hbm.at[pl.ds(base_w + c * _CHUNK, _CHUNK)],
                         xbuf.at[slot], xsems[slot])

    def wait_x(c, slot):
        pltpu.make_async_copy(x_hbm.at[pl.ds(base_w + c * _CHUNK, _CHUNK)],
                              xbuf.at[slot], xsems[slot]).wait()

    def start_out(c, slot):
        pltpu.async_copy(rowbuf.at[slot],
                         out_hbm.at[pl.ds(base_w + c * _CHUNK, _CHUNK)],
                         osems[slot])

    def wait_out(c, slot):
        pltpu.make_async_copy(rowbuf.at[slot],
                              out_hbm.at[pl.ds(base_w + c * _CHUNK, _CHUNK)],
                              osems[slot]).wait()

    def accumulate(slot):
        def row_body(r, c2):
            for j in range(_D // _LANES):
                off = j * _LANES
                v = xbuf[slot, r, pl.ds(off, _LANES)]
                plsc.addupdate(rowbuf.at[slot, r, pl.ds(off, _LANES)], v)
            return c2
        lax.fori_loop(0, _CHUNK, row_body, 0)

    for c in range(_LEAD):
        start_gather(c, c % _NBUF)
        start_x(c, c % _NBUF)

    def outer(i0, carry):
        for u in range(_NBUF):
            j = i0 * _NBUF + u
            wait_gather(j, u)
            wait_x(j, u)
            nxt_slot = (u + _LEAD) % _NBUF

            @pl.when(j >= _NBUF - _LEAD)
            def _drain():
                wait_out(j - (_NBUF - _LEAD), nxt_slot)

            @pl.when(j + _LEAD < n_chunks)
            def _issue():
                start_gather(j + _LEAD, nxt_slot)
                start_x(j + _LEAD, nxt_slot)

            accumulate(u)
            start_out(j, u)
        return carry

    lax.fori_loop(0, n_chunks // _NBUF, outer, 0)

    for c in range(n_chunks - (_NBUF - _LEAD), n_chunks):
        wait_out(c, c % _NBUF)


def kernel(x, positions, pos_table):
    b, s, d = x.shape
    n = b * s
    x2 = x.reshape(n, d)
    pos = positions.reshape(n).astype(jnp.int32)
    mesh = plsc.VectorSubcoreMesh(core_axis_name="c", subcore_axis_name="s")
    f = pl.kernel(
        _pe_body,
        mesh=mesh,
        out_type=jax.ShapeDtypeStruct((n, d), jnp.float32),
        scratch_types=[
            pltpu.VMEM((n // _NW,), jnp.int32),
            pltpu.VMEM((_NBUF, _CHUNK, d), jnp.float32),
            pltpu.VMEM((_NBUF, _CHUNK, d), jnp.float32),
        ] + [pltpu.SemaphoreType.DMA] * (3 * _NBUF),
    )
    out = f(x2, pos, pos_table)
    return out.reshape(b, s, d)


# speedup vs baseline: 1.5384x; 1.0104x over previous
import jax
import jax.numpy as jnp
from jax import lax
from jax.experimental import pallas as pl
from jax.experimental.pallas import tpu as pltpu
from jax.experimental.pallas import tpu_sc as plsc

_D = 1024
_LANES = 16
_NC, _NS = 2, 16
_NW = _NC * _NS
_CHUNK = 8
_NBUF = 4
_LEAD = 3


def _pe_body(x_hbm, pos_hbm, tab_hbm, out_hbm,
             idx_all, rowbuf, xbuf, *sems):
    gsems = sems[0:_NBUF]
    xsems = sems[_NBUF:2 * _NBUF]
    osems = sems[2 * _NBUF:3 * _NBUF]
    wid = lax.axis_index("s") * _NC + lax.axis_index("c")
    n_rows = pos_hbm.shape[0]
    rows_per_w = n_rows // _NW
    base_w = wid * rows_per_w
    n_chunks = rows_per_w // _CHUNK

    pltpu.sync_copy(pos_hbm.at[pl.ds(base_w, rows_per_w)], idx_all)

    def start_gather(c, slot):
        idx = idx_all.at[pl.ds(c * _CHUNK, _CHUNK)]
        pltpu.async_copy(tab_hbm.at[idx], rowbuf.at[slot], gsems[slot])

    def wait_gather(c, slot):
        idx = idx_all.at[pl.ds(c * _CHUNK, _CHUNK)]
        pltpu.make_async_copy(tab_hbm.at[idx], rowbuf.at[slot],
                              gsems[slot]).wait()

    def start_x(c, slot):
        pltpu.async_copy(x_hbm.at[pl.ds(base_w + c * _CHUNK, _CHUNK)],
                         xbuf.at[slot], xsems[slot])

    def wait_x(c, slot):
        pltpu.make_async_copy(x_hbm.at[pl.ds(base_w + c * _CHUNK, _CHUNK)],
                              xbuf.at[slot], xsems[slot]).wait()

    def start_out(c, slot):
        pltpu.async_copy(rowbuf.at[slot],
                         out_hbm.at[pl.ds(base_w + c * _CHUNK, _CHUNK)],
                         osems[slot])

    def wait_out(c, slot):
        pltpu.make_async_copy(rowbuf.at[slot],
                              out_hbm.at[pl.ds(base_w + c * _CHUNK, _CHUNK)],
                              osems[slot]).wait()

    def accumulate(slot):
        def row_body(r, c2):
            for j in range(_D // _LANES):
                off = j * _LANES
                v = xbuf[slot, r, pl.ds(off, _LANES)]
                plsc.addupdate(rowbuf.at[slot, r, pl.ds(off, _LANES)], v)
            return c2
        lax.fori_loop(0, _CHUNK, row_body, 0)

    for c in range(_LEAD):
        start_gather(c, c % _NBUF)
        start_x(c, c % _NBUF)

    def outer(i0, carry):
        for u in range(_NBUF):
            j = i0 * _NBUF + u
            wait_gather(j, u)
            wait_x(j, u)
            nxt_slot = (u + _LEAD) % _NBUF

            @pl.when(j >= _NBUF - _LEAD)
            def _drain():
                wait_out(j - (_NBUF - _LEAD), nxt_slot)

            @pl.when(j + _LEAD < n_chunks)
            def _issue():
                start_gather(j + _LEAD, nxt_slot)
                start_x(j + _LEAD, nxt_slot)

            accumulate(u)
            start_out(j, u)
        return carry

    lax.fori_loop(0, n_chunks // _NBUF, outer, 0)

    for c in range(n_chunks - (_NBUF - _LEAD), n_chunks):
        wait_out(c, c % _NBUF)


def kernel(x, positions, pos_table):
    b, s, d = x.shape
    n = b * s
    x2 = x.reshape(n, d)
    pos = positions.reshape(n).astype(jnp.int32)
    mesh = plsc.VectorSubcoreMesh(core_axis_name="c", subcore_axis_name="s")
    f = pl.kernel(
        _pe_body,
        mesh=mesh,
        out_type=jax.ShapeDtypeStruct((n, d), jnp.float32),
        scratch_types=[
            pltpu.VMEM((n // _NW,), jnp.int32),
            pltpu.VMEM((_NBUF, _CHUNK, d), jnp.float32),
            pltpu.VMEM((_NBUF, _CHUNK, d), jnp.float32),
        ] + [pltpu.SemaphoreType.DMA] * (3 * _NBUF),
    )
    out = f(x2, pos, pos_table)
    return out.reshape(b, s, d)
